# Initial kernel scaffold; baseline (speedup 1.0000x reference)
import jax
import jax.numpy as jnp
from jax.experimental import pallas as pl


def kernel(image, feat, w_rpn, b_rpn, w_cls, b_cls, w_reg, b_reg):
    raise NotImplementedError("write your pallas kernel here")


# speedup vs baseline: 71.4082x; 71.4082x over previous
import functools
import math

import jax
import jax.numpy as jnp
from jax import lax
from jax.experimental import pallas as pl
from jax.experimental.pallas import tpu as pltpu

C = 128
H = W = 128
HP = 130
NPIX = H * HP
NPIX_PAD = 16904
A = 9
PRE_TOPK = 10000
NBOX = 10240
POST_TOPK = 2000
NMS_THRESH = 0.7
LOGK = math.log(1000.0 / 16)
BP = 1040


def _head_kernel(x_ref, w3_ref, b_ref, w1_ref, b1_ref, anc_ref, out_ref):
    i = pl.program_id(0)
    base = i * BP
    acc = jnp.zeros((BP, C), jnp.float32)
    for dy in range(3):
        for dx in range(3):
            xs = x_ref[pl.ds(base + dy * HP + dx, BP), :]
            acc = acc + jnp.dot(xs, w3_ref[dy * 3 + dx],
                                preferred_element_type=jnp.float32)
    f = jnp.maximum(acc + b_ref[0, :], 0.0)
    cr = jnp.dot(f, w1_ref[...], preferred_element_type=jnp.float32) + b1_ref[0, :]
    pieces = [jax.nn.sigmoid(cr[:, 0:A])]
    for a in range(A):
        d = cr[:, A + 4 * a: A + 4 * a + 4]
        w = anc_ref[0, a]
        h = anc_ref[1, a]
        cx = anc_ref[2, a]
        cy = anc_ref[3, a]
        dw = jnp.minimum(d[:, 2:3], LOGK)
        dh = jnp.minimum(d[:, 3:4], LOGK)
        pcx = d[:, 0:1] * w + cx
        pcy = d[:, 1:2] * h + cy
        pw = jnp.exp(dw) * w
        ph = jnp.exp(dh) * h
        x1 = jnp.clip(pcx - 0.5 * pw, 0.0, float(W))
        y1 = jnp.clip(pcy - 0.5 * ph, 0.0, float(H))
        x2 = jnp.clip(pcx + 0.5 * pw, 0.0, float(W))
        y2 = jnp.clip(pcy + 0.5 * ph, 0.0, float(H))
        pieces.append(jnp.concatenate([x1, y1, x2, y2], axis=1))
    out_ref[...] = jnp.concatenate(pieces, axis=1)


def _nms_kernel(boxes_ref, boxes_s_ref, scores_s_ref, ob_ref, os_ref,
                alive_ref, area_ref):
    ob_ref[...] = jnp.zeros((POST_TOPK, 4), jnp.float32)
    os_ref[...] = jnp.zeros((POST_TOPK, 1), jnp.float32)
    idxs = lax.broadcasted_iota(jnp.int32, (1, NBOX), 1)
    alive_ref[...] = jnp.where(idxs < PRE_TOPK, 1.0, 0.0)
    x1 = boxes_ref[0:1, :]
    y1 = boxes_ref[1:2, :]
    x2 = boxes_ref[2:3, :]
    y2 = boxes_ref[3:4, :]
    area_ref[...] = (x2 - x1) * (y2 - y1)

    def cond(carry):
        return carry[1]

    def body(carry):
        c, _ = carry
        alive = alive_ref[...]
        pidx = jnp.min(jnp.where(alive > 0, idxs, jnp.int32(NBOX)))
        pv = boxes_s_ref[pl.ds(pidx, 1), :]
        x1p = pv[0:1, 0:1]
        y1p = pv[0:1, 1:2]
        x2p = pv[0:1, 2:3]
        y2p = pv[0:1, 3:4]
        ap = (x2p - x1p) * (y2p - y1p)
        xx1 = jnp.maximum(x1p, x1)
        yy1 = jnp.maximum(y1p, y1)
        xx2 = jnp.minimum(x2p, x2)
        yy2 = jnp.minimum(y2p, y2)
        inter = jnp.maximum(xx2 - xx1, 0.0) * jnp.maximum(yy2 - yy1, 0.0)
        iou = inter / (ap + area_ref[...] - inter + 1e-9)
        newalive = jnp.where((iou > NMS_THRESH) | (idxs == pidx), 0.0, alive)
        alive_ref[...] = newalive
        os_ref[pl.ds(c, 1), :] = scores_s_ref[pl.ds(pidx, 1), :]
        ob_ref[pl.ds(c, 1), :] = pv
        go = (jnp.max(newalive) > 0.0) & (c + 1 < POST_TOPK)
        return (c + 1, go)

    lax.while_loop(cond, body, (jnp.int32(0), jnp.bool_(True)))


def _base_anchors():
    scales = jnp.asarray((128.0, 256.0, 512.0), dtype=jnp.float32)
    ar = jnp.asarray((0.5, 1.0, 2.0), dtype=jnp.float32)
    h_ratios = jnp.sqrt(ar)
    w_ratios = 1.0 / h_ratios
    ws = (w_ratios[:, None] * scales[None, :]).reshape(-1)
    hs = (h_ratios[:, None] * scales[None, :]).reshape(-1)
    return jnp.round(jnp.stack([-ws, -hs, ws, hs], axis=1) / 2.0)


def _regress(pred, anchors):
    w = anchors[:, 2] - anchors[:, 0]
    h = anchors[:, 3] - anchors[:, 1]
    cx = anchors[:, 0] + 0.5 * w
    cy = anchors[:, 1] + 0.5 * h
    dw = jnp.minimum(pred[:, 2], LOGK)
    dh = jnp.minimum(pred[:, 3], LOGK)
    pcx = pred[:, 0] * w + cx
    pcy = pred[:, 1] * h + cy
    pw = jnp.exp(dw) * w
    ph = jnp.exp(dh) * h
    x1 = jnp.clip(pcx - 0.5 * pw, 0, W)
    y1 = jnp.clip(pcy - 0.5 * ph, 0, H)
    x2 = jnp.clip(pcx + 0.5 * pw, 0, W)
    y2 = jnp.clip(pcy + 0.5 * ph, 0, H)
    return jnp.stack([x1, y1, x2, y2], axis=1)


@jax.jit
def kernel(image, feat, w_rpn, b_rpn, w_cls, b_cls, w_reg, b_reg):
    del feat
    x = jnp.transpose(image[0], (1, 2, 0))
    x = jnp.pad(x, ((1, 1), (1, 1), (0, 0))).reshape(HP * HP, C)
    x = jnp.pad(x, ((0, NPIX_PAD - HP * HP), (0, 0)))
    w3 = jnp.transpose(w_rpn, (2, 3, 1, 0)).reshape(9, C, C)
    w1 = jnp.concatenate([w_cls[:, :, 0, 0], w_reg[:, :, 0, 0]], axis=0).T
    b1 = jnp.concatenate([b_cls, b_reg]).reshape(1, 45)
    anchors = _base_anchors()
    anc = jnp.stack([anchors[:, 2] - anchors[:, 0],
                     anchors[:, 3] - anchors[:, 1],
                     anchors[:, 0] + 0.5 * (anchors[:, 2] - anchors[:, 0]),
                     anchors[:, 1] + 0.5 * (anchors[:, 3] - anchors[:, 1])])
    anc = jnp.pad(anc, ((0, 0), (0, 16 - A)))

    s45 = pl.pallas_call(
        _head_kernel,
        grid=(NPIX // BP,),
        in_specs=[
            pl.BlockSpec((NPIX_PAD, C), lambda i: (0, 0)),
            pl.BlockSpec((9, C, C), lambda i: (0, 0, 0)),
            pl.BlockSpec((1, C), lambda i: (0, 0)),
            pl.BlockSpec((C, 45), lambda i: (0, 0)),
            pl.BlockSpec((1, 45), lambda i: (0, 0)),
            pl.BlockSpec(memory_space=pltpu.SMEM),
        ],
        out_specs=pl.BlockSpec((BP, 45), lambda i: (i, 0)),
        out_shape=jax.ShapeDtypeStruct((NPIX, 45), jnp.float32),
    )(x, w3, b_rpn.reshape(1, C), w1, b1, anc)

    s3 = s45.reshape(H, HP, 45)[:, :W, :]
    border_scores = jax.nn.sigmoid(b_cls)
    border_boxes = _regress(b_reg.reshape(A, 4), _base_anchors())
    scores_full = jnp.broadcast_to(border_scores, (HP, HP, A))
    scores_full = lax.dynamic_update_slice(scores_full, s3[:, :, :A], (1, 1, 0))
    scores_full = scores_full.reshape(-1)
    boxes_full = jnp.broadcast_to(border_boxes, (HP, HP, A, 4))
    boxes_full = lax.dynamic_update_slice(
        boxes_full, s3[:, :, A:].reshape(H, W, A, 4), (1, 1, 0, 0))
    boxes_full = boxes_full.reshape(-1, 4)

    top_scores, top_idx = lax.top_k(scores_full, PRE_TOPK)
    props = boxes_full[top_idx]
    boxes_s = jnp.pad(props, ((0, NBOX - PRE_TOPK), (0, 0)))
    boxes_t = boxes_s.T
    scores_s = jnp.pad(top_scores, (0, NBOX - PRE_TOPK)).reshape(NBOX, 1)

    out_boxes, out_scores = pl.pallas_call(
        _nms_kernel,
        out_shape=[jax.ShapeDtypeStruct((POST_TOPK, 4), jnp.float32),
                   jax.ShapeDtypeStruct((POST_TOPK, 1), jnp.float32)],
        scratch_shapes=[pltpu.VMEM((1, NBOX), jnp.float32),
                        pltpu.VMEM((1, NBOX), jnp.float32)],
    )(boxes_t, boxes_s, scores_s)
    return out_boxes, out_scores.reshape(POST_TOPK)


# speedup vs baseline: 145.9792x; 2.0443x over previous
import functools
import math

import jax
import jax.numpy as jnp
from jax import lax
from jax.experimental import pallas as pl
from jax.experimental.pallas import tpu as pltpu
from jax.experimental.pallas import tpu_sc as plsc

C = 128
H = W = 128
HP = 130
PLANE = H * W
XROWS = HP * W
A = 9
PRE_TOPK = 10000
NBOX = 10240
POST_TOPK = 2000
NMS_THRESH = 0.7
LOGK = math.log(1000.0 / 16)
BN = 2048

NC, NS = 2, 16
NW = NC * NS
CHUNK = NBOX // NW
SUB = 64


def _head_kernel(x0_ref, x1_ref, x2_ref, w3_ref, b_ref, w1_ref, b1_ref,
                 anc_ref, s_ref, bp_ref):
    i = pl.program_id(0)
    base = i * BN
    acc = jnp.zeros((C, BN), jnp.float32)
    xrefs = (x0_ref, x1_ref, x2_ref)
    for dy in range(3):
        for dx in range(3):
            xs = xrefs[dx][:, pl.ds(base + dy * W, BN)]
            acc = acc + jnp.dot(w3_ref[dy * 3 + dx], xs,
                                preferred_element_type=jnp.float32)
    f = jnp.maximum(acc + b_ref[...], 0.0)
    cr = jnp.dot(w1_ref[...], f, preferred_element_type=jnp.float32) + b1_ref[...]
    s_ref[...] = jax.nn.sigmoid(cr[0:A, :])
    for a in range(A):
        d0 = cr[A + 4 * a: A + 4 * a + 1, :]
        d1 = cr[A + 4 * a + 1: A + 4 * a + 2, :]
        d2 = cr[A + 4 * a + 2: A + 4 * a + 3, :]
        d3 = cr[A + 4 * a + 3: A + 4 * a + 4, :]
        w = anc_ref[0, a]
        h = anc_ref[1, a]
        cx = anc_ref[2, a]
        cy = anc_ref[3, a]
        dw = jnp.minimum(d2, LOGK)
        dh = jnp.minimum(d3, LOGK)
        pcx = d0 * w + cx
        pcy = d1 * h + cy
        pw = jnp.exp(dw) * w
        ph = jnp.exp(dh) * h
        vals = (jnp.clip(pcx - 0.5 * pw, 0.0, float(W)),
                jnp.clip(pcy - 0.5 * ph, 0.0, float(H)),
                jnp.clip(pcx + 0.5 * pw, 0.0, float(W)),
                jnp.clip(pcy + 0.5 * ph, 0.0, float(H)))
        for k in range(4):
            bp_ref[pl.ds((k * A + a) * PLANE + base, BN)] = vals[k].reshape(BN)


def _sc_gather_kernel(bp_hbm, e4_hbm, e4i_hbm, bt_hbm, bs_hbm,
                      idx_v, idx2_v, g4_v, bs_v, sem):
    wid = lax.axis_index("s") * NC + lax.axis_index("c")
    base = wid * CHUNK
    for k in range(4):
        pltpu.sync_copy(e4_hbm.at[pl.ds(k * NBOX + base, CHUNK)],
                        idx_v.at[pl.ds(k * CHUNK, CHUNK)])
    pltpu.sync_copy(e4i_hbm.at[pl.ds(base * 4, CHUNK * 4)], idx2_v)
    copies = []
    for s in range(4 * CHUNK // SUB):
        copies.append(pltpu.async_copy(
            bp_hbm.at[idx_v.at[pl.ds(s * SUB, SUB)]],
            g4_v.at[pl.ds(s * SUB, SUB)], sem))
        copies.append(pltpu.async_copy(
            bp_hbm.at[idx2_v.at[pl.ds(s * SUB, SUB)]],
            bs_v.at[pl.ds(s * SUB, SUB)], sem))
    for cp in copies:
        cp.wait()
    for k in range(4):
        pltpu.sync_copy(g4_v.at[pl.ds(k * CHUNK, CHUNK)],
                        bt_hbm.at[pl.ds(k * NBOX + base, CHUNK)])
    pltpu.sync_copy(bs_v, bs_hbm.at[pl.ds(base * 4, CHUNK * 4)])


def _nms_kernel(boxes_ref, boxes_s_ref, scores_s_ref, ob_ref, os_ref,
                alive_ref, area_ref):
    ob_ref[...] = jnp.zeros((POST_TOPK, 4), jnp.float32)
    os_ref[...] = jnp.zeros((POST_TOPK, 1), jnp.float32)
    idxs = lax.broadcasted_iota(jnp.int32, (1, NBOX), 1)
    alive_ref[...] = jnp.where(idxs < PRE_TOPK, 1.0, 0.0)
    x1 = boxes_ref[0:1, :]
    y1 = boxes_ref[1:2, :]
    x2 = boxes_ref[2:3, :]
    y2 = boxes_ref[3:4, :]
    area_ref[...] = (x2 - x1) * (y2 - y1)

    def cond(carry):
        return carry[1]

    def body(carry):
        c, _ = carry
        alive = alive_ref[...]
        pidx = jnp.min(jnp.where(alive > 0, idxs, jnp.int32(NBOX)))
        pv = boxes_s_ref[pl.ds(pidx, 1), :]
        x1p = pv[0:1, 0:1]
        y1p = pv[0:1, 1:2]
        x2p = pv[0:1, 2:3]
        y2p = pv[0:1, 3:4]
        ap = (x2p - x1p) * (y2p - y1p)
        xx1 = jnp.maximum(x1p, x1)
        yy1 = jnp.maximum(y1p, y1)
        xx2 = jnp.minimum(x2p, x2)
        yy2 = jnp.minimum(y2p, y2)
        inter = jnp.maximum(xx2 - xx1, 0.0) * jnp.maximum(yy2 - yy1, 0.0)
        iou = inter / (ap + area_ref[...] - inter + 1e-9)
        newalive = jnp.where((iou > NMS_THRESH) | (idxs == pidx), 0.0, alive)
        alive_ref[...] = newalive
        os_ref[pl.ds(c, 1), :] = scores_s_ref[pl.ds(pidx, 1), :]
        ob_ref[pl.ds(c, 1), :] = pv
        go = (jnp.max(newalive) > 0.0) & (c + 1 < POST_TOPK)
        return (c + 1, go)

    lax.while_loop(cond, body, (jnp.int32(0), jnp.bool_(True)))


def _base_anchors():
    scales = jnp.asarray((128.0, 256.0, 512.0), dtype=jnp.float32)
    ar = jnp.asarray((0.5, 1.0, 2.0), dtype=jnp.float32)
    h_ratios = jnp.sqrt(ar)
    w_ratios = 1.0 / h_ratios
    ws = (w_ratios[:, None] * scales[None, :]).reshape(-1)
    hs = (h_ratios[:, None] * scales[None, :]).reshape(-1)
    return jnp.round(jnp.stack([-ws, -hs, ws, hs], axis=1) / 2.0)


def _regress(pred, anchors):
    w = anchors[:, 2] - anchors[:, 0]
    h = anchors[:, 3] - anchors[:, 1]
    cx = anchors[:, 0] + 0.5 * w
    cy = anchors[:, 1] + 0.5 * h
    dw = jnp.minimum(pred[:, 2], LOGK)
    dh = jnp.minimum(pred[:, 3], LOGK)
    pcx = pred[:, 0] * w + cx
    pcy = pred[:, 1] * h + cy
    pw = jnp.exp(dw) * w
    ph = jnp.exp(dh) * h
    x1 = jnp.clip(pcx - 0.5 * pw, 0, W)
    y1 = jnp.clip(pcy - 0.5 * ph, 0, H)
    x2 = jnp.clip(pcx + 0.5 * pw, 0, W)
    y2 = jnp.clip(pcy + 0.5 * ph, 0, H)
    return jnp.stack([x1, y1, x2, y2], axis=1)


@jax.jit
def kernel(image, feat, w_rpn, b_rpn, w_cls, b_cls, w_reg, b_reg):
    del feat
    xp = jnp.pad(image[0], ((0, 0), (1, 1), (1, 1)))
    xs3 = [xp[:, :, dx:dx + W].reshape(C, XROWS) for dx in range(3)]
    w3 = jnp.transpose(w_rpn, (2, 3, 0, 1)).reshape(9, C, C)
    w1 = jnp.concatenate([w_cls[:, :, 0, 0], w_reg[:, :, 0, 0]], axis=0)
    b1 = jnp.concatenate([b_cls, b_reg]).reshape(45, 1)
    anchors = _base_anchors()
    anc = jnp.stack([anchors[:, 2] - anchors[:, 0],
                     anchors[:, 3] - anchors[:, 1],
                     anchors[:, 0] + 0.5 * (anchors[:, 2] - anchors[:, 0]),
                     anchors[:, 1] + 0.5 * (anchors[:, 3] - anchors[:, 1])])
    anc = jnp.pad(anc, ((0, 0), (0, 16 - A)))

    scores9, bplanar = pl.pallas_call(
        _head_kernel,
        grid=(PLANE // BN,),
        in_specs=[
            pl.BlockSpec((C, XROWS), lambda i: (0, 0)),
            pl.BlockSpec((C, XROWS), lambda i: (0, 0)),
            pl.BlockSpec((C, XROWS), lambda i: (0, 0)),
            pl.BlockSpec((9, C, C), lambda i: (0, 0, 0)),
            pl.BlockSpec((C, 1), lambda i: (0, 0)),
            pl.BlockSpec((45, C), lambda i: (0, 0)),
            pl.BlockSpec((45, 1), lambda i: (0, 0)),
            pl.BlockSpec(memory_space=pltpu.SMEM),
        ],
        out_specs=[pl.BlockSpec((A, BN), lambda i: (0, i)),
                   pl.BlockSpec((36 * PLANE,), lambda i: (0,))],
        out_shape=[jax.ShapeDtypeStruct((A, PLANE), jnp.float32),
                   jax.ShapeDtypeStruct((36 * PLANE,), jnp.float32)],
    )(xs3[0], xs3[1], xs3[2], w3, b_rpn.reshape(C, 1), w1, b1, anc)

    s9 = scores9.reshape(A, H, W)
    border_scores = jax.nn.sigmoid(b_cls)
    border_boxes = _regress(b_reg.reshape(A, 4), _base_anchors())
    full = jnp.broadcast_to(border_scores[:, None, None], (A, HP, HP))
    full = lax.dynamic_update_slice(full, s9, (0, 1, 1))
    scores_full = jnp.transpose(full.reshape(A, HP * HP), (1, 0)).reshape(-1)

    top_scores, top_idx = lax.top_k(scores_full, PRE_TOPK)
    n = jnp.pad(top_idx, (0, NBOX - PRE_TOPK)).astype(jnp.int32)
    q = n // A
    a = n - q * A
    yc = q // HP
    xc = q - yc * HP
    interior = (yc >= 1) & (yc <= H) & (xc >= 1) & (xc <= W)
    p = (yc - 1) * W + (xc - 1)
    base_j = jnp.where(interior, a * PLANE + p, 36 * PLANE + a)
    step_j = jnp.where(interior, A * PLANE, 16)
    ks = jnp.arange(4, dtype=jnp.int32)
    e4 = (base_j[None, :] + ks[:, None] * step_j[None, :]).reshape(-1)
    e4i = (base_j[:, None] + ks[None, :] * step_j[:, None]).reshape(-1)
    bb = jnp.zeros((4, 16), jnp.float32).at[:, :A].set(border_boxes.T)
    bp_ext = jnp.concatenate([bplanar, bb.reshape(-1)])

    mesh = plsc.VectorSubcoreMesh(core_axis_name="c", subcore_axis_name="s")
    bt_flat, bs_flat = pl.kernel(
        _sc_gather_kernel,
        out_type=(jax.ShapeDtypeStruct((4 * NBOX,), jnp.float32),
                  jax.ShapeDtypeStruct((4 * NBOX,), jnp.float32)),
        mesh=mesh,
        scratch_types=[
            pltpu.VMEM((4 * CHUNK,), jnp.int32),
            pltpu.VMEM((4 * CHUNK,), jnp.int32),
            pltpu.VMEM((4 * CHUNK,), jnp.float32),
            pltpu.VMEM((4 * CHUNK,), jnp.float32),
            pltpu.SemaphoreType.DMA,
        ],
    )(bp_ext, e4, e4i)

    boxes_t = bt_flat.reshape(4, NBOX)
    boxes_s = bs_flat.reshape(NBOX, 4)
    scores_s = jnp.pad(top_scores, (0, NBOX - PRE_TOPK)).reshape(NBOX, 1)

    out_boxes, out_scores = pl.pallas_call(
        _nms_kernel,
        out_shape=[jax.ShapeDtypeStruct((POST_TOPK, 4), jnp.float32),
                   jax.ShapeDtypeStruct((POST_TOPK, 1), jnp.float32)],
        scratch_shapes=[pltpu.VMEM((1, NBOX), jnp.float32),
                        pltpu.VMEM((1, NBOX), jnp.float32)],
    )(boxes_t, boxes_s, scores_s)
    return out_boxes, out_scores.reshape(POST_TOPK)
